# baseline (speedup 1.0000x reference)
import functools

import jax
import jax.numpy as jnp
from jax import lax
from jax.experimental import pallas as pl
from jax.experimental.pallas import tpu as pltpu
from jax.experimental.pallas import tpu_sc as plsc

NUM_USERS = 1000000
NUM_ITEMS = 100000
EMB = 64
SHW = 32
B = 16384
PK = 128

NC = 2
NS = 16
NW = NC * NS
PW = B // NW
L = 16
CH = 64
NCH = PW // CH


def _body(u2d_r, i2d_r, ush_r, ish_r, up_r, ip_r, mean_r, out_r,
          uidx, iidx, up_v, ip_v, ush_v, ish_v, outb, mean_v, sem):
    wid = lax.axis_index("s") * NC + lax.axis_index("c")
    base = wid * PW

    pltpu.sync_copy(u2d_r.at[pl.ds(wid * NCH, NCH)], uidx)
    pltpu.sync_copy(i2d_r.at[pl.ds(wid * NCH, NCH)], iidx)
    pltpu.sync_copy(mean_r, mean_v)
    mv = mean_v[...]
    iota = lax.iota(jnp.int32, L)

    for j in range(NCH):
        cps = [
            pltpu.async_copy(up_r.at[uidx.at[j]], up_v, sem),
            pltpu.async_copy(ip_r.at[iidx.at[j]], ip_v, sem),
            pltpu.async_copy(ush_r.at[pl.ds(base + j * CH, CH)], ush_v, sem),
            pltpu.async_copy(ish_r.at[pl.ds(base + j * CH, CH)], ish_v, sem),
        ]
        for cp in cps:
            cp.wait()

        def group(g, carry, j=j):
            res = mv
            for k in range(L):
                r = g * L + k
                acc = up_v[r, pl.ds(0, L)] * ip_v[r, pl.ds(0, L)]
                for t in range(1, EMB // L):
                    acc += (up_v[r, pl.ds(t * L, L)]
                            * ip_v[r, pl.ds(t * L, L)])
                for t in range(SHW // L):
                    acc += (ush_v[r, pl.ds(t * L, L)]
                            * ip_v[r, pl.ds(EMB + t * L, L)])
                    acc += (ish_v[r, pl.ds(t * L, L)]
                            * up_v[r, pl.ds(EMB + t * L, L)])
                bpair = (up_v[r, pl.ds(EMB + SHW, L)]
                         + ip_v[r, pl.ds(EMB + SHW, L)])
                res += jnp.where(iota == k, jnp.sum(acc) + bpair[0], 0.0)
            outb[pl.ds(j * CH + g * L, L)] = res
            return carry

        lax.fori_loop(0, CH // L, group, 0)

    pltpu.sync_copy(outb, out_r.at[pl.ds(base, PW)])


@functools.partial(jax.jit, static_argnames=())
def kernel(u_id, i_id, UserShadow, ItemShadow, user_emb_w, user_bias_w,
           item_emb_w, item_bias_w, shadow_u_w, shadow_i_w, mean):
    u2d = u_id.astype(jnp.int32).reshape(B // CH, CH)
    i2d = i_id.astype(jnp.int32).reshape(B // CH, CH)
    mean16 = jnp.broadcast_to(mean.astype(jnp.float32), (L,))
    f32 = jnp.float32
    upad = jnp.zeros((NUM_USERS, PK - EMB - SHW - 1), f32)
    ipad = jnp.zeros((NUM_ITEMS, PK - EMB - SHW - 1), f32)
    upack = jnp.concatenate([user_emb_w, shadow_u_w, user_bias_w, upad], 1)
    ipack = jnp.concatenate([item_emb_w, shadow_i_w, item_bias_w, ipad], 1)

    mesh = plsc.VectorSubcoreMesh(core_axis_name="c", subcore_axis_name="s")
    run = pl.kernel(
        _body,
        out_type=jax.ShapeDtypeStruct((B,), f32),
        mesh=mesh,
        compiler_params=pltpu.CompilerParams(needs_layout_passes=False),
        scratch_types=[
            pltpu.VMEM((NCH, CH), jnp.int32),
            pltpu.VMEM((NCH, CH), jnp.int32),
            pltpu.VMEM((CH, PK), f32),
            pltpu.VMEM((CH, PK), f32),
            pltpu.VMEM((CH, SHW), f32),
            pltpu.VMEM((CH, SHW), f32),
            pltpu.VMEM((PW,), f32),
            pltpu.VMEM((L,), f32),
            pltpu.SemaphoreType.DMA,
        ],
    )
    return run(u2d, i2d, UserShadow, ItemShadow, upack, ipack, mean16)


# speedup vs baseline: 1.4516x; 1.4516x over previous
import functools

import jax
import jax.numpy as jnp
from jax import lax
from jax.experimental import pallas as pl
from jax.experimental.pallas import tpu as pltpu
from jax.experimental.pallas import tpu_sc as plsc

NUM_USERS = 1000000
NUM_ITEMS = 100000
EMB = 64
SHW = 32
B = 16384

NC = 2
NS = 16
NW = NC * NS
PW = B // NW
L = 16
NG = PW // L
CHB = 128
NCHB = PW // CHB


def _dot_body(u2d_r, i2d_r, ush_r, ish_r, ue_r, ie_r, su_r, si_r, out_r,
              uidx, iidx, ue_v, ie_v, su_v, si_v, ush_v, ish_v, outb, sem):
    wid = lax.axis_index("s") * NC + lax.axis_index("c")
    base = wid * PW

    pltpu.sync_copy(u2d_r.at[pl.ds(wid * 4, 4)], uidx)
    pltpu.sync_copy(i2d_r.at[pl.ds(wid * 4, 4)], iidx)
    iota = lax.iota(jnp.int32, L)

    def group(g, carry):
        uv = uidx[g // 8, pl.ds((g % 8) * L, L)]
        iv = iidx[g // 8, pl.ds((g % 8) * L, L)]
        usl = lax.shift_right_logical(uv, 3)
        isl = lax.shift_right_logical(iv, 3)
        ru_v = uv & 7
        ri_v = iv & 7
        cps = [
            pltpu.async_copy(ush_r.at[pl.ds(base + g * L, L)], ush_v, sem),
            pltpu.async_copy(ish_r.at[pl.ds(base + g * L, L)], ish_v, sem),
        ]
        for k in range(L):
            uo = pl.multiple_of(usl[k] * 8, 8)
            io = pl.multiple_of(isl[k] * 8, 8)
            cps.append(pltpu.async_copy(
                ue_r.at[pl.ds(uo, 8), :], ue_v.at[k], sem))
            cps.append(pltpu.async_copy(
                ie_r.at[pl.ds(io, 8), :], ie_v.at[k], sem))
            cps.append(pltpu.async_copy(
                su_r.at[pl.ds(uo, 8), :], su_v.at[k], sem))
            cps.append(pltpu.async_copy(
                si_r.at[pl.ds(io, 8), :], si_v.at[k], sem))
        for cp in cps:
            cp.wait()

        res = jnp.zeros((L,), jnp.float32)
        for k in range(L):
            ru = ru_v[k]
            ri = ri_v[k]
            acc = ue_v[k, ru, pl.ds(0, L)] * ie_v[k, ri, pl.ds(0, L)]
            for t in range(1, EMB // L):
                acc += (ue_v[k, ru, pl.ds(t * L, L)]
                        * ie_v[k, ri, pl.ds(t * L, L)])
            for t in range(SHW // L):
                acc += ush_v[k, pl.ds(t * L, L)] * si_v[k, ri, pl.ds(t * L, L)]
                acc += ish_v[k, pl.ds(t * L, L)] * su_v[k, ru, pl.ds(t * L, L)]
            res += jnp.where(iota == k, jnp.sum(acc), 0.0)
        outb[pl.ds(g * L, L)] = res
        return carry

    lax.fori_loop(0, NG, group, 0)
    pltpu.sync_copy(outb, out_r.at[pl.ds(base, PW)])


def _bias_body(u2d_r, i2d_r, part_r, ub_r, ib_r, mean_r, out_r,
               uidx, iidx, bu_v, bi_v, partb, mean_v, sem):
    wid = lax.axis_index("s") * NC + lax.axis_index("c")
    base = wid * PW

    pltpu.sync_copy(u2d_r.at[pl.ds(wid * 4, 4)], uidx)
    pltpu.sync_copy(i2d_r.at[pl.ds(wid * 4, 4)], iidx)
    pltpu.sync_copy(part_r.at[pl.ds(base, PW)], partb)
    pltpu.sync_copy(mean_r, mean_v)
    mv = mean_v[...]

    for j in range(NCHB):
        cps = [
            pltpu.async_copy(ub_r.at[uidx.at[j]], bu_v, sem),
            pltpu.async_copy(ib_r.at[iidx.at[j]], bi_v, sem),
        ]
        for cp in cps:
            cp.wait()

        def grp(g, carry, j=j):
            o = j * CHB + g * L
            out_v = (partb[pl.ds(o, L)] + bu_v[pl.ds(g * L, L)]
                     + bi_v[pl.ds(g * L, L)] + mv)
            partb[pl.ds(o, L)] = out_v
            return carry

        lax.fori_loop(0, CHB // L, grp, 0)

    pltpu.sync_copy(partb, out_r.at[pl.ds(base, PW)])


@functools.partial(jax.jit, static_argnames=())
def kernel(u_id, i_id, UserShadow, ItemShadow, user_emb_w, user_bias_w,
           item_emb_w, item_bias_w, shadow_u_w, shadow_i_w, mean):
    u2d = u_id.astype(jnp.int32).reshape(B // 128, 128)
    i2d = i_id.astype(jnp.int32).reshape(B // 128, 128)
    mean16 = jnp.broadcast_to(mean.astype(jnp.float32), (L,))
    ub1 = user_bias_w.reshape(NUM_USERS)
    ib1 = item_bias_w.reshape(NUM_ITEMS)

    f32 = jnp.float32
    mesh = plsc.VectorSubcoreMesh(core_axis_name="c", subcore_axis_name="s")
    dots = pl.kernel(
        _dot_body,
        out_type=jax.ShapeDtypeStruct((B,), f32),
        mesh=mesh,
        compiler_params=pltpu.CompilerParams(needs_layout_passes=False),
        scratch_types=[
            pltpu.VMEM((4, 128), jnp.int32),
            pltpu.VMEM((4, 128), jnp.int32),
            pltpu.VMEM((L, 8, EMB), f32),
            pltpu.VMEM((L, 8, EMB), f32),
            pltpu.VMEM((L, 8, SHW), f32),
            pltpu.VMEM((L, 8, SHW), f32),
            pltpu.VMEM((L, SHW), f32),
            pltpu.VMEM((L, SHW), f32),
            pltpu.VMEM((PW,), f32),
            pltpu.SemaphoreType.DMA,
        ],
    )
    partial_out = dots(u2d, i2d, UserShadow, ItemShadow,
                       user_emb_w, item_emb_w, shadow_u_w, shadow_i_w)

    biases = pl.kernel(
        _bias_body,
        out_type=jax.ShapeDtypeStruct((B,), f32),
        mesh=mesh,
        compiler_params=pltpu.CompilerParams(
            needs_layout_passes=False, use_tc_tiling_on_sc=False),
        scratch_types=[
            pltpu.VMEM((4, 128), jnp.int32),
            pltpu.VMEM((4, 128), jnp.int32),
            pltpu.VMEM((CHB,), f32),
            pltpu.VMEM((CHB,), f32),
            pltpu.VMEM((PW,), f32),
            pltpu.VMEM((L,), f32),
            pltpu.SemaphoreType.DMA,
        ],
    )
    return biases(u2d, i2d, partial_out, ub1, ib1, mean16)
